# Initial kernel scaffold; baseline (speedup 1.0000x reference)
import jax
import jax.numpy as jnp
from jax.experimental import pallas as pl


def kernel(x, edge_attr, params, edge_index, batch):
    raise NotImplementedError("write your pallas kernel here")


# speedup vs baseline: 10.1959x; 10.1959x over previous
import functools

import jax
import jax.numpy as jnp
from jax import lax
from jax.experimental import pallas as pl
from jax.experimental.pallas import tpu as pltpu
from jax.experimental.pallas import tpu_sc as plsc

N = 10000
E = 320000
D = 128
H = 128
G = 256
C = 10
NP = 10240
NACC = 10016
NW = 32
EW_A = E // NW
EW_B = E // 16

_mesh = plsc.VectorSubcoreMesh(core_axis_name="c", subcore_axis_name="s")


def _sc_prep_body(col_hbm, ew_hbm, degp_hbm, colidx_hbm,
                  colv, ewv, cmv, deg0, deg1, deg2):
    cid = lax.axis_index("c")
    sid = lax.axis_index("s")
    wid = sid * 2 + cid
    base = wid * EW_A
    degs = [deg0, deg1, deg2]

    pltpu.sync_copy(col_hbm.at[pl.ds(base, EW_A)], colv)

    def zero(j, _):
        z = jnp.zeros((16,), jnp.float32)
        deg0[pl.ds(j * 16, 16)] = z
        deg1[pl.ds(j * 16, 16)] = z
        deg2[pl.ds(j * 16, 16)] = z
        return 0
    lax.fori_loop(0, NP // 16, zero, 0)

    lane = lax.iota(jnp.int32, 16)
    for b in range(3):
        pltpu.sync_copy(ew_hbm.at[pl.ds(b * E + base, EW_A)], ewv)

        def step(j, _, b=b):
            sl = pl.ds(j * 16, 16)
            cv = colv[sl]
            ev = ewv[sl]
            cmv[sl] = jnp.where(ev > 0.5, cv, N + lane)
            plsc.addupdate_scatter(degs[b], [cv], ev)
            return 0
        lax.fori_loop(0, EW_A // 16, step, 0)
        pltpu.sync_copy(cmv, colidx_hbm.at[pl.ds(b * E + base, EW_A)])
        pltpu.sync_copy(degs[b], degp_hbm.at[pl.ds((wid * 3 + b) * NP, NP)])


_sc_prep = pl.kernel(
    _sc_prep_body,
    out_type=(
        jax.ShapeDtypeStruct((NW * 3 * NP,), jnp.float32),
        jax.ShapeDtypeStruct((3 * E,), jnp.int32),
    ),
    mesh=_mesh,
    scratch_types=[
        pltpu.VMEM((EW_A,), jnp.int32),
        pltpu.VMEM((EW_A,), jnp.float32),
        pltpu.VMEM((EW_A,), jnp.int32),
        pltpu.VMEM((NP,), jnp.float32),
        pltpu.VMEM((NP,), jnp.float32),
        pltpu.VMEM((NP,), jnp.float32),
    ],
    compiler_params=pltpu.CompilerParams(needs_layout_passes=False),
)


def _sc_agg_body(hb0, hb1, hb2, row_hbm, colidx_hbm, s0, s1, s2,
                 rbuf, abuf, cbuf, msg, rbuf2, abuf2, cbuf2, msg2,
                 acc0, acc1, acc2):
    cid = lax.axis_index("c")
    sid = lax.axis_index("s")
    hbs = [hb0, hb1, hb2]
    accs = [acc0, acc1, acc2]
    outs = [s0, s1, s2]

    def zrow(j, _):
        z = jnp.zeros((16,), jnp.float32)
        for k in range(4):
            msg[j, pl.ds(k * 16, 16)] = z
        return 0
    lax.fori_loop(0, 128, zrow, 0)
    zbase = sid * 626
    for b in range(3):
        for k in range(4):
            pltpu.sync_copy(msg,
                            accs[b].at[pl.ds(zbase + k * 128, 128)])
        pltpu.sync_copy(msg.at[pl.ds(0, 114)],
                        accs[b].at[pl.ds(zbase + 512, 114)])
    plsc.subcore_barrier()

    off = cid * N
    tbase = sid * EW_B

    def chunk(base, kk, rb, ab, cb, mb):
        pltpu.sync_copy(row_hbm.at[pl.ds(base, kk)], rb)
        for j in range(kk // 16):
            sl = pl.ds(j * 16, 16)
            ab[sl] = rb[sl] + off
        for b in range(3):
            pltpu.sync_copy(colidx_hbm.at[pl.ds(b * E + base, kk)], cb)
            pltpu.sync_copy(hbs[b].at[ab], mb)
            pltpu.sync_copy(mb, accs[b].at[cb], add=True)

    def step(i, _):
        chunk(tbase + i * 128, 128, rbuf, abuf, cbuf, msg)
        return 0
    lax.fori_loop(0, 156, step, 0)
    chunk(tbase + 156 * 128, 32, rbuf2, abuf2, cbuf2, msg2)

    plsc.subcore_barrier()
    obase = sid * 626
    for b in range(3):
        pltpu.sync_copy(accs[b].at[pl.ds(obase, 626)],
                        outs[b].at[cid, pl.ds(obase, 626)])


_sc_agg = pl.kernel(
    _sc_agg_body,
    out_type=(
        jax.ShapeDtypeStruct((2, NACC, 64), jnp.float32),
        jax.ShapeDtypeStruct((2, NACC, 64), jnp.float32),
        jax.ShapeDtypeStruct((2, NACC, 64), jnp.float32),
    ),
    mesh=_mesh,
    scratch_types=[
        pltpu.VMEM((128,), jnp.int32),
        pltpu.VMEM((128,), jnp.int32),
        pltpu.VMEM((128,), jnp.int32),
        pltpu.VMEM((128, 64), jnp.float32),
        pltpu.VMEM((32,), jnp.int32),
        pltpu.VMEM((32,), jnp.int32),
        pltpu.VMEM((32,), jnp.int32),
        pltpu.VMEM((32, 64), jnp.float32),
        pltpu.VMEM_SHARED((NACC, 64), jnp.float32),
        pltpu.VMEM_SHARED((NACC, 64), jnp.float32),
        pltpu.VMEM_SHARED((NACC, 64), jnp.float32),
    ],
    compiler_params=pltpu.CompilerParams(use_tc_tiling_on_sc=False),
)


def _dinv_body(degp_ref, dinv_ref):
    deg = jnp.sum(degp_ref[...], axis=0)
    dinv_ref[...] = jnp.where(deg > 0, lax.rsqrt(jnp.maximum(deg, 1e-12)), 0.0)


def _tc_dinv(degp):
    return pl.pallas_call(
        _dinv_body,
        out_shape=jax.ShapeDtypeStruct((3, NP), jnp.float32),
    )(degp)


def _scale_body(x_ref, dinv_ref, h0_ref, h1_ref, h2_ref):
    x = x_ref[...]
    outs = [h0_ref, h1_ref, h2_ref]
    for b in range(3):
        db = dinv_ref[:, b][:, None]
        outs[b][0] = db * x[:, :64]
        outs[b][1] = db * x[:, 64:]


def _tc_scale(x, dinv):
    blk = 1000
    grid = N // blk
    out_specs = tuple(
        pl.BlockSpec((2, blk, 64), lambda i: (0, i, 0)) for _ in range(3))
    return pl.pallas_call(
        _scale_body,
        grid=(grid,),
        in_specs=[
            pl.BlockSpec((blk, D), lambda i: (i, 0)),
            pl.BlockSpec((blk, 3), lambda i: (i, 0)),
        ],
        out_specs=out_specs,
        out_shape=tuple(
            jax.ShapeDtypeStruct((2, N, 64), jnp.float32) for _ in range(3)),
    )(x, dinv)


def _layer_a_body(h_ref, s0_ref, s1_ref, s2_ref, dinv_ref,
                  w_ref, bias_ref, u_ref, stats_ref):
    srefs = [s0_ref, s1_ref, s2_ref]
    acc = None
    for b in range(3):
        sb = jnp.concatenate([srefs[b][0], srefs[b][1]], axis=1)
        pre = dinv_ref[:, b][:, None] * sb
        ob = jnp.maximum(
            jnp.dot(pre, w_ref[b], preferred_element_type=jnp.float32)
            + bias_ref[b, :][None, :], 0.0)
        acc = ob if acc is None else acc + ob
    xi = jnp.maximum(
        jnp.dot(h_ref[...], w_ref[3], preferred_element_type=jnp.float32)
        + bias_ref[3, :][None, :], 0.0)
    u = acc + xi
    u_ref[...] = u

    @pl.when(pl.program_id(0) == 0)
    def _():
        stats_ref[...] = jnp.zeros_like(stats_ref)

    stats_ref[0:1, :] += jnp.sum(u, axis=0, keepdims=True)
    stats_ref[1:2, :] += jnp.sum(u * u, axis=0, keepdims=True)


def _tc_layer_a(h, s0, s1, s2, dinv, w4, b4):
    blk = 1000
    grid = N // blk
    sspec = pl.BlockSpec((2, blk, 64), lambda i: (0, i, 0))
    return pl.pallas_call(
        _layer_a_body,
        grid=(grid,),
        in_specs=[
            pl.BlockSpec((blk, D), lambda i: (i, 0)),
            sspec, sspec, sspec,
            pl.BlockSpec((blk, 3), lambda i: (i, 0)),
            pl.BlockSpec((4, D, H), lambda i: (0, 0, 0)),
            pl.BlockSpec((4, H), lambda i: (0, 0)),
        ],
        out_specs=(
            pl.BlockSpec((blk, H), lambda i: (i, 0)),
            pl.BlockSpec((2, H), lambda i: (0, 0)),
        ),
        out_shape=(
            jax.ShapeDtypeStruct((N, H), jnp.float32),
            jax.ShapeDtypeStruct((2, H), jnp.float32),
        ),
    )(h, s0, s1, s2, dinv, w4, b4)


def _layer_b_body(u_ref, stats_ref, gb_ref, dinv_ref, h_ref,
                  h0_ref, h1_ref, h2_ref, *, make_hb):
    mean = stats_ref[0:1, :] / N
    var = stats_ref[1:2, :] / N - mean * mean
    rstd = lax.rsqrt(var + 1e-5)
    hn = (u_ref[...] - mean) * rstd * gb_ref[0:1, :] + gb_ref[1:2, :]
    h_ref[...] = hn
    if make_hb:
        outs = [h0_ref, h1_ref, h2_ref]
        for b in range(3):
            db = dinv_ref[:, b][:, None]
            outs[b][0] = db * hn[:, :64]
            outs[b][1] = db * hn[:, 64:]


def _tc_layer_b(u, stats, gb, dinv, make_hb):
    blk = 1000
    grid = N // blk
    body = functools.partial(_layer_b_body, make_hb=make_hb)
    if make_hb:
        out_specs = (
            pl.BlockSpec((blk, H), lambda i: (i, 0)),
            pl.BlockSpec((2, blk, 64), lambda i: (0, i, 0)),
            pl.BlockSpec((2, blk, 64), lambda i: (0, i, 0)),
            pl.BlockSpec((2, blk, 64), lambda i: (0, i, 0)),
        )
        out_shape = (
            jax.ShapeDtypeStruct((N, H), jnp.float32),
            jax.ShapeDtypeStruct((2, N, 64), jnp.float32),
            jax.ShapeDtypeStruct((2, N, 64), jnp.float32),
            jax.ShapeDtypeStruct((2, N, 64), jnp.float32),
        )
    else:
        def body(u_ref, stats_ref, gb_ref, dinv_ref, h_ref):
            _layer_b_body(u_ref, stats_ref, gb_ref, dinv_ref, h_ref,
                          None, None, None, make_hb=False)
        out_specs = pl.BlockSpec((blk, H), lambda i: (i, 0))
        out_shape = jax.ShapeDtypeStruct((N, H), jnp.float32)
    return pl.pallas_call(
        body,
        grid=(grid,),
        in_specs=[
            pl.BlockSpec((blk, H), lambda i: (i, 0)),
            pl.BlockSpec((2, H), lambda i: (0, 0)),
            pl.BlockSpec((2, H), lambda i: (0, 0)),
            pl.BlockSpec((blk, 3), lambda i: (i, 0)),
        ],
        out_specs=out_specs,
        out_shape=out_shape,
    )(u, stats, gb, dinv)


def _pool_body(h_ref, batch_ref, pooled_ref, counts_ref):
    bt = batch_ref[0, 0, :]
    gi = lax.broadcasted_iota(jnp.int32, (G, bt.shape[0]), 0)
    oh = (gi == bt[None, :]).astype(jnp.float32)

    @pl.when(pl.program_id(0) == 0)
    def _():
        pooled_ref[...] = jnp.zeros_like(pooled_ref)
        counts_ref[...] = jnp.zeros_like(counts_ref)

    pooled_ref[...] += jnp.dot(oh, h_ref[...],
                               preferred_element_type=jnp.float32)
    counts_ref[...] += jnp.sum(oh, axis=1)[None, :]


def _tc_pool(h, batch3d):
    blk = 1000
    grid = N // blk
    return pl.pallas_call(
        _pool_body,
        grid=(grid,),
        in_specs=[
            pl.BlockSpec((blk, H), lambda i: (i, 0)),
            pl.BlockSpec((1, 1, blk), lambda i: (i, 0, 0)),
        ],
        out_specs=(
            pl.BlockSpec((G, H), lambda i: (0, 0)),
            pl.BlockSpec((1, G), lambda i: (0, 0)),
        ),
        out_shape=(
            jax.ShapeDtypeStruct((G, H), jnp.float32),
            jax.ShapeDtypeStruct((1, G), jnp.float32),
        ),
    )(h, batch3d)


def _mlp_body(pooled_ref, counts_ref, w1a_ref, w1b_ref, b1_ref,
              w2_ref, b2_ref, out_ref):
    cnt = counts_ref[0, :][:, None] / 40.0
    z = (jnp.dot(pooled_ref[...], w1a_ref[...],
                 preferred_element_type=jnp.float32)
         + cnt * w1b_ref[0:1, :] + b1_ref[0:1, :])
    z = jnp.maximum(z, 0.0)
    out_ref[...] = (jnp.dot(z, w2_ref[...],
                            preferred_element_type=jnp.float32)
                    + b2_ref[0:1, :])


def _tc_mlp(pooled, counts, w1a, w1b, b1, w2, b2):
    return pl.pallas_call(
        _mlp_body,
        out_shape=jax.ShapeDtypeStruct((G, C), jnp.float32),
    )(pooled, counts, w1a, w1b, b1, w2, b2)


def kernel(x, edge_attr, params, edge_index, batch):
    row = edge_index[0]
    col = edge_index[1]
    ewT = jnp.transpose(edge_attr[:, :3]).reshape(-1)

    degp, colidx = _sc_prep(col, ewT)
    dinv_p = _tc_dinv(degp.reshape(NW, 3, NP))
    dinv = jnp.transpose(dinv_p)

    hb = _tc_scale(x, dinv)
    hb = [a.reshape(2 * N, 64) for a in hb]

    h = x
    for li, lyr in enumerate(params["layers"]):
        s0, s1, s2 = _sc_agg(hb[0], hb[1], hb[2], row, colidx)
        w4 = jnp.stack([lyr["Ws"], lyr["Wd"], lyr["Wt"], lyr["Wi"]])
        b4 = jnp.stack([lyr["bs"], lyr["bd"], lyr["bt"], lyr["bi"]])
        u, stats = _tc_layer_a(h, s0, s1, s2, dinv, w4, b4)
        gb = jnp.stack([lyr["g"], lyr["be"]])
        if li == 0:
            h, h0, h1, h2 = _tc_layer_b(u, stats, gb, dinv, True)
            hb = [h0.reshape(2 * N, 64), h1.reshape(2 * N, 64),
                  h2.reshape(2 * N, 64)]
        else:
            h = _tc_layer_b(u, stats, gb, dinv, False)

    batch3d = batch.reshape(10, 1, N // 10)
    pooled, counts = _tc_pool(h, batch3d)
    w1a = params["fc1_W"][:H, :]
    w1b = params["fc1_W"][H:, :]
    return _tc_mlp(pooled, counts, w1a, w1b,
                   params["fc1_b"][None, :], params["fc2_W"],
                   params["fc2_b"][None, :])


# speedup vs baseline: 20.2389x; 1.9850x over previous
import functools

import jax
import jax.numpy as jnp
from jax import lax
from jax.experimental import pallas as pl
from jax.experimental.pallas import tpu as pltpu
from jax.experimental.pallas import tpu_sc as plsc

N = 10000
E = 320000
D = 128
H = 128
G = 256
C = 10
NP = 10240
NACC = 10016
NW = 32
EW_A = E // NW
EW_B = E // 16

_mesh = plsc.VectorSubcoreMesh(core_axis_name="c", subcore_axis_name="s")


def _sc_prep_body(col_hbm, row_hbm, ew_hbm, degp_hbm, colidx_hbm, rowadj_hbm,
                  colv, ewv, cmv, deg0, deg1, deg2):
    cid = lax.axis_index("c")
    sid = lax.axis_index("s")
    wid = sid * 2 + cid
    base = wid * EW_A
    degs = [deg0, deg1, deg2]

    pltpu.sync_copy(row_hbm.at[pl.ds(base, EW_A)], colv)

    def radj(j, _):
        sl = pl.ds(j * 16, 16)
        cmv[sl] = colv[sl] + N
        return 0
    lax.fori_loop(0, EW_A // 16, radj, 0)
    pltpu.sync_copy(colv, rowadj_hbm.at[pl.ds(base, EW_A)])
    pltpu.sync_copy(cmv, rowadj_hbm.at[pl.ds(E + base, EW_A)])

    pltpu.sync_copy(col_hbm.at[pl.ds(base, EW_A)], colv)

    def zero(j, _):
        z = jnp.zeros((16,), jnp.float32)
        deg0[pl.ds(j * 16, 16)] = z
        deg1[pl.ds(j * 16, 16)] = z
        deg2[pl.ds(j * 16, 16)] = z
        return 0
    lax.fori_loop(0, NP // 16, zero, 0)

    lane = lax.iota(jnp.int32, 16)
    for b in range(3):
        pltpu.sync_copy(ew_hbm.at[pl.ds(b * E + base, EW_A)], ewv)

        def step(j, _, b=b):
            sl = pl.ds(j * 16, 16)
            cv = colv[sl]
            ev = ewv[sl]
            cmv[sl] = jnp.where(ev > 0.5, cv, N + lane)
            plsc.addupdate_scatter(degs[b], [cv], ev)
            return 0
        lax.fori_loop(0, EW_A // 16, step, 0)
        pltpu.sync_copy(cmv, colidx_hbm.at[pl.ds(b * E + base, EW_A)])
        pltpu.sync_copy(degs[b], degp_hbm.at[pl.ds((wid * 3 + b) * NP, NP)])


_sc_prep = pl.kernel(
    _sc_prep_body,
    out_type=(
        jax.ShapeDtypeStruct((NW * 3 * NP,), jnp.float32),
        jax.ShapeDtypeStruct((3 * E,), jnp.int32),
        jax.ShapeDtypeStruct((2 * E,), jnp.int32),
    ),
    mesh=_mesh,
    scratch_types=[
        pltpu.VMEM((EW_A,), jnp.int32),
        pltpu.VMEM((EW_A,), jnp.float32),
        pltpu.VMEM((EW_A,), jnp.int32),
        pltpu.VMEM((NP,), jnp.float32),
        pltpu.VMEM((NP,), jnp.float32),
        pltpu.VMEM((NP,), jnp.float32),
    ],
    compiler_params=pltpu.CompilerParams(needs_layout_passes=False),
)


def _sc_agg_body(hb0, hb1, hb2, rowadj_hbm, colidx_hbm, s0, s1, s2,
                 ab0, ab1,
                 cb00, cb01, cb02, cb10, cb11, cb12, cb20, cb21, cb22,
                 m00, m01, m02, m10, m11, m12,
                 rbuf2, cbuf2, msg2, zbuf,
                 acc0,
                 si0, si1, sg0, sg1, ss0, ss1):
    cid = lax.axis_index("c")
    sid = lax.axis_index("s")
    hbs = [hb0, hb1, hb2]
    accs = [acc0]
    outs = [s0, s1, s2]
    abuf = [ab0, ab1]
    cbuf = [[cb00, cb01, cb02], [cb10, cb11, cb12], [cb20, cb21, cb22]]
    msg = [[m00, m01, m02], [m10, m11, m12]]
    sem_i = [si0, si1]
    sem_g = [sg0, sg1]
    sem_s = [ss0, ss1]

    zbase = sid * 626
    tbase = sid * EW_B
    roff = cid * E + tbase

    def zrow(j, _):
        z = jnp.zeros((16,), jnp.float32)
        for k in range(4):
            zbuf[j, pl.ds(k * 16, 16)] = z
        return 0
    lax.fori_loop(0, 128, zrow, 0)

    def zero_acc(a):
        for k in range(4):
            pltpu.sync_copy(zbuf, a.at[pl.ds(zbase + k * 128, 128)])
        pltpu.sync_copy(zbuf.at[pl.ds(0, 114)],
                        a.at[pl.ds(zbase + 512, 114)])

    def pipeline(bset, amap):
        def idx_start(c, p, r):
            pltpu.async_copy(rowadj_hbm.at[pl.ds(roff + c * 128, 128)],
                             abuf[p], sem_i[p])
            for b in bset:
                pltpu.async_copy(
                    colidx_hbm.at[pl.ds(b * E + tbase + c * 128, 128)],
                    cbuf[r][b], sem_i[p])

        def idx_wait(c, p, r):
            pltpu.make_async_copy(rowadj_hbm.at[pl.ds(roff + c * 128, 128)],
                                  abuf[p], sem_i[p]).wait()
            for b in bset:
                pltpu.make_async_copy(
                    colidx_hbm.at[pl.ds(b * E + tbase + c * 128, 128)],
                    cbuf[r][b], sem_i[p]).wait()

        def gat_start(q):
            for b in bset:
                pltpu.async_copy(hbs[b].at[abuf[q]], msg[q][b], sem_g[q])

        def gat_wait(q):
            for b in bset:
                pltpu.make_async_copy(hbs[b].at[abuf[q]], msg[q][b],
                                      sem_g[q]).wait()

        def sct_start(p, r):
            for b in bset:
                pltpu.async_copy(msg[p][b], accs[amap[b]].at[cbuf[r][b]],
                                 sem_s[p], add=True)

        def sct_wait(p, r):
            for b in bset:
                pltpu.make_async_copy(msg[p][b], accs[amap[b]].at[cbuf[r][b]],
                                      sem_s[p]).wait()

        def full_body(c, p, q, r):
            sct_wait(q, r)
            idx_wait(c - 1, q, (r + 2) % 3)
            gat_start(q)
            gat_wait(p)
            sct_start(p, (r + 1) % 3)
            idx_start(c, p, r)

        idx_start(0, 0, 0)
        idx_wait(0, 0, 0)
        gat_start(0)
        idx_start(1, 1, 1)
        idx_wait(1, 1, 1)
        gat_start(1)
        gat_wait(0)
        sct_start(0, 0)
        idx_start(2, 0, 2)

        def steady(g, _):
            c = 3 + 6 * g
            for o in range(6):
                full_body(c + o, (3 + o) % 2, (o + 2) % 2, o % 3)
            return 0
        lax.fori_loop(0, 25, steady, 0)
        full_body(153, 1, 0, 0)
        full_body(154, 0, 1, 1)
        full_body(155, 1, 0, 2)

        sct_wait(1, 0)
        idx_wait(155, 1, 2)
        gat_start(1)
        gat_wait(0)
        sct_start(0, 1)
        gat_wait(1)
        sct_start(1, 2)
        sct_wait(0, 1)
        sct_wait(1, 2)

        pltpu.sync_copy(rowadj_hbm.at[pl.ds(roff + 19968, 32)], rbuf2)
        for b in bset:
            pltpu.sync_copy(colidx_hbm.at[pl.ds(b * E + tbase + 19968, 32)],
                            cbuf2)
            pltpu.sync_copy(hbs[b].at[rbuf2], msg2)
            pltpu.sync_copy(msg2, accs[amap[b]].at[cbuf2], add=True)

    for b in range(3):
        zero_acc(acc0)
        plsc.subcore_barrier()
        pipeline([b], {b: 0})
        plsc.subcore_barrier()
        pltpu.sync_copy(acc0.at[pl.ds(zbase, 626)],
                        outs[b].at[cid, pl.ds(zbase, 626)])


_sc_agg = pl.kernel(
    _sc_agg_body,
    out_type=(
        jax.ShapeDtypeStruct((2, NACC, 64), jnp.float32),
        jax.ShapeDtypeStruct((2, NACC, 64), jnp.float32),
        jax.ShapeDtypeStruct((2, NACC, 64), jnp.float32),
    ),
    mesh=_mesh,
    scratch_types=(
        [pltpu.VMEM((128,), jnp.int32) for _ in range(11)]
        + [pltpu.VMEM((128, 64), jnp.float32) for _ in range(6)]
        + [pltpu.VMEM((32,), jnp.int32), pltpu.VMEM((32,), jnp.int32),
           pltpu.VMEM((32, 64), jnp.float32),
           pltpu.VMEM((128, 64), jnp.float32)]
        + [pltpu.VMEM_SHARED((NACC, 64), jnp.float32)]
        + [pltpu.SemaphoreType.DMA for _ in range(6)]
    ),
    compiler_params=pltpu.CompilerParams(use_tc_tiling_on_sc=False),
)


def _dinv_body(degp_ref, dinv_ref):
    deg = jnp.sum(degp_ref[...], axis=0)
    dinv_ref[...] = jnp.where(deg > 0, lax.rsqrt(jnp.maximum(deg, 1e-12)), 0.0)


def _tc_dinv(degp):
    return pl.pallas_call(
        _dinv_body,
        out_shape=jax.ShapeDtypeStruct((3, NP), jnp.float32),
    )(degp)


def _scale_body(x_ref, dinv_ref, h0_ref, h1_ref, h2_ref):
    x = x_ref[...]
    outs = [h0_ref, h1_ref, h2_ref]
    for b in range(3):
        db = dinv_ref[:, b][:, None]
        outs[b][0] = db * x[:, :64]
        outs[b][1] = db * x[:, 64:]


def _tc_scale(x, dinv):
    blk = 1000
    grid = N // blk
    out_specs = tuple(
        pl.BlockSpec((2, blk, 64), lambda i: (0, i, 0)) for _ in range(3))
    return pl.pallas_call(
        _scale_body,
        grid=(grid,),
        in_specs=[
            pl.BlockSpec((blk, D), lambda i: (i, 0)),
            pl.BlockSpec((blk, 3), lambda i: (i, 0)),
        ],
        out_specs=out_specs,
        out_shape=tuple(
            jax.ShapeDtypeStruct((2, N, 64), jnp.float32) for _ in range(3)),
    )(x, dinv)


def _layer_a_body(h_ref, s0_ref, s1_ref, s2_ref, dinv_ref,
                  w_ref, bias_ref, u_ref, stats_ref):
    srefs = [s0_ref, s1_ref, s2_ref]
    acc = None
    for b in range(3):
        sb = jnp.concatenate([srefs[b][0], srefs[b][1]], axis=1)
        pre = dinv_ref[:, b][:, None] * sb
        ob = jnp.maximum(
            jnp.dot(pre, w_ref[b], preferred_element_type=jnp.float32)
            + bias_ref[b, :][None, :], 0.0)
        acc = ob if acc is None else acc + ob
    xi = jnp.maximum(
        jnp.dot(h_ref[...], w_ref[3], preferred_element_type=jnp.float32)
        + bias_ref[3, :][None, :], 0.0)
    u = acc + xi
    u_ref[...] = u

    @pl.when(pl.program_id(0) == 0)
    def _():
        stats_ref[...] = jnp.zeros_like(stats_ref)

    stats_ref[0:1, :] += jnp.sum(u, axis=0, keepdims=True)
    stats_ref[1:2, :] += jnp.sum(u * u, axis=0, keepdims=True)


def _tc_layer_a(h, s0, s1, s2, dinv, w4, b4):
    blk = 1000
    grid = N // blk
    sspec = pl.BlockSpec((2, blk, 64), lambda i: (0, i, 0))
    return pl.pallas_call(
        _layer_a_body,
        grid=(grid,),
        in_specs=[
            pl.BlockSpec((blk, D), lambda i: (i, 0)),
            sspec, sspec, sspec,
            pl.BlockSpec((blk, 3), lambda i: (i, 0)),
            pl.BlockSpec((4, D, H), lambda i: (0, 0, 0)),
            pl.BlockSpec((4, H), lambda i: (0, 0)),
        ],
        out_specs=(
            pl.BlockSpec((blk, H), lambda i: (i, 0)),
            pl.BlockSpec((2, H), lambda i: (0, 0)),
        ),
        out_shape=(
            jax.ShapeDtypeStruct((N, H), jnp.float32),
            jax.ShapeDtypeStruct((2, H), jnp.float32),
        ),
    )(h, s0, s1, s2, dinv, w4, b4)


def _layer_b_body(u_ref, stats_ref, gb_ref, dinv_ref, h_ref,
                  h0_ref, h1_ref, h2_ref, *, make_hb):
    mean = stats_ref[0:1, :] / N
    var = stats_ref[1:2, :] / N - mean * mean
    rstd = lax.rsqrt(var + 1e-5)
    hn = (u_ref[...] - mean) * rstd * gb_ref[0:1, :] + gb_ref[1:2, :]
    h_ref[...] = hn
    if make_hb:
        outs = [h0_ref, h1_ref, h2_ref]
        for b in range(3):
            db = dinv_ref[:, b][:, None]
            outs[b][0] = db * hn[:, :64]
            outs[b][1] = db * hn[:, 64:]


def _tc_layer_b(u, stats, gb, dinv, make_hb):
    blk = 1000
    grid = N // blk
    body = functools.partial(_layer_b_body, make_hb=make_hb)
    if make_hb:
        out_specs = (
            pl.BlockSpec((blk, H), lambda i: (i, 0)),
            pl.BlockSpec((2, blk, 64), lambda i: (0, i, 0)),
            pl.BlockSpec((2, blk, 64), lambda i: (0, i, 0)),
            pl.BlockSpec((2, blk, 64), lambda i: (0, i, 0)),
        )
        out_shape = (
            jax.ShapeDtypeStruct((N, H), jnp.float32),
            jax.ShapeDtypeStruct((2, N, 64), jnp.float32),
            jax.ShapeDtypeStruct((2, N, 64), jnp.float32),
            jax.ShapeDtypeStruct((2, N, 64), jnp.float32),
        )
    else:
        def body(u_ref, stats_ref, gb_ref, dinv_ref, h_ref):
            _layer_b_body(u_ref, stats_ref, gb_ref, dinv_ref, h_ref,
                          None, None, None, make_hb=False)
        out_specs = pl.BlockSpec((blk, H), lambda i: (i, 0))
        out_shape = jax.ShapeDtypeStruct((N, H), jnp.float32)
    return pl.pallas_call(
        body,
        grid=(grid,),
        in_specs=[
            pl.BlockSpec((blk, H), lambda i: (i, 0)),
            pl.BlockSpec((2, H), lambda i: (0, 0)),
            pl.BlockSpec((2, H), lambda i: (0, 0)),
            pl.BlockSpec((blk, 3), lambda i: (i, 0)),
        ],
        out_specs=out_specs,
        out_shape=out_shape,
    )(u, stats, gb, dinv)


def _pool_body(h_ref, batch_ref, pooled_ref, counts_ref):
    bt = batch_ref[0, 0, :]
    gi = lax.broadcasted_iota(jnp.int32, (G, bt.shape[0]), 0)
    oh = (gi == bt[None, :]).astype(jnp.float32)

    @pl.when(pl.program_id(0) == 0)
    def _():
        pooled_ref[...] = jnp.zeros_like(pooled_ref)
        counts_ref[...] = jnp.zeros_like(counts_ref)

    pooled_ref[...] += jnp.dot(oh, h_ref[...],
                               preferred_element_type=jnp.float32)
    counts_ref[...] += jnp.sum(oh, axis=1)[None, :]


def _tc_pool(h, batch3d):
    blk = 1000
    grid = N // blk
    return pl.pallas_call(
        _pool_body,
        grid=(grid,),
        in_specs=[
            pl.BlockSpec((blk, H), lambda i: (i, 0)),
            pl.BlockSpec((1, 1, blk), lambda i: (i, 0, 0)),
        ],
        out_specs=(
            pl.BlockSpec((G, H), lambda i: (0, 0)),
            pl.BlockSpec((1, G), lambda i: (0, 0)),
        ),
        out_shape=(
            jax.ShapeDtypeStruct((G, H), jnp.float32),
            jax.ShapeDtypeStruct((1, G), jnp.float32),
        ),
    )(h, batch3d)


def _mlp_body(pooled_ref, counts_ref, w1a_ref, w1b_ref, b1_ref,
              w2_ref, b2_ref, out_ref):
    cnt = counts_ref[0, :][:, None] / 40.0
    z = (jnp.dot(pooled_ref[...], w1a_ref[...],
                 preferred_element_type=jnp.float32)
         + cnt * w1b_ref[0:1, :] + b1_ref[0:1, :])
    z = jnp.maximum(z, 0.0)
    out_ref[...] = (jnp.dot(z, w2_ref[...],
                            preferred_element_type=jnp.float32)
                    + b2_ref[0:1, :])


def _tc_mlp(pooled, counts, w1a, w1b, b1, w2, b2):
    return pl.pallas_call(
        _mlp_body,
        out_shape=jax.ShapeDtypeStruct((G, C), jnp.float32),
    )(pooled, counts, w1a, w1b, b1, w2, b2)


def kernel(x, edge_attr, params, edge_index, batch):
    row = edge_index[0]
    col = edge_index[1]
    ewT = jnp.transpose(edge_attr[:, :3]).reshape(-1)

    degp, colidx, rowadj = _sc_prep(col, row, ewT)
    dinv_p = _tc_dinv(degp.reshape(NW, 3, NP))
    dinv = jnp.transpose(dinv_p)

    hb = _tc_scale(x, dinv)
    hb = [a.reshape(2 * N, 64) for a in hb]

    h = x
    for li, lyr in enumerate(params["layers"]):
        s0, s1, s2 = _sc_agg(hb[0], hb[1], hb[2], rowadj, colidx)
        w4 = jnp.stack([lyr["Ws"], lyr["Wd"], lyr["Wt"], lyr["Wi"]])
        b4 = jnp.stack([lyr["bs"], lyr["bd"], lyr["bt"], lyr["bi"]])
        u, stats = _tc_layer_a(h, s0, s1, s2, dinv, w4, b4)
        gb = jnp.stack([lyr["g"], lyr["be"]])
        if li == 0:
            h, h0, h1, h2 = _tc_layer_b(u, stats, gb, dinv, True)
            hb = [h0.reshape(2 * N, 64), h1.reshape(2 * N, 64),
                  h2.reshape(2 * N, 64)]
        else:
            h = _tc_layer_b(u, stats, gb, dinv, False)

    batch3d = batch.reshape(10, 1, N // 10)
    pooled, counts = _tc_pool(h, batch3d)
    w1a = params["fc1_W"][:H, :]
    w1b = params["fc1_W"][H:, :]
    return _tc_mlp(pooled, counts, w1a, w1b,
                   params["fc1_b"][None, :], params["fc2_W"],
                   params["fc2_b"][None, :])
